# Initial kernel scaffold; baseline (speedup 1.0000x reference)
import jax
import jax.numpy as jnp
from jax.experimental import pallas as pl


def kernel(x, edge_index, W1, b1, W2, b2, W3, b3, Wd, bd, Wc, bc, Wt, bt, Wa, ba):
    raise NotImplementedError("write your pallas kernel here")


# speedup vs baseline: 24.4280x; 24.4280x over previous
import functools

import jax
import jax.numpy as jnp
from jax import lax
from jax.experimental import pallas as pl
from jax.experimental.pallas import tpu as pltpu
from jax.experimental.pallas import tpu_sc as plsc

F32 = jnp.float32
I32 = jnp.int32

_NC = 2
_NS = 16
_L = 16
_ROW = 128
_CROWS = 4
_PAIR = 2 * _CROWS

_BN = 4000


def _mesh():
    return plsc.VectorSubcoreMesh(core_axis_name="c", subcore_axis_name="s")


def _deg_body(dst_hbm, out_hbm, deg_v, chunk_v):
    c = lax.axis_index("c")
    t = lax.axis_index("s")
    w = c * _NS + t
    n = out_hbm.shape[1]
    n_pad = deg_v.shape[0]
    e_tile = dst_hbm.shape[0] // (_NC * _NS)
    ch = chunk_v.shape[0]
    zeros = jnp.zeros((_L,), F32)
    ones = jnp.ones((_L,), F32)

    def zbody(i, carry):
        deg_v[pl.ds(i * _L, _L)] = zeros
        return carry

    lax.fori_loop(0, n_pad // _L, zbody, 0, unroll=8)

    base = w * e_tile
    for k in range(e_tile // ch):
        pltpu.sync_copy(dst_hbm.at[pl.ds(base + k * ch, ch)], chunk_v)

        def ebody(j, carry):
            idx = chunk_v[pl.ds(j * _L, _L)]
            plsc.addupdate_scatter(deg_v, [idx], ones)
            return carry

        lax.fori_loop(0, ch // _L, ebody, 0, unroll=8)

    pltpu.sync_copy(deg_v, out_hbm.at[w])


def _npad(n):
    sl = (n // (_NC * _NS) + 16) // 16 * 16
    return sl, sl * _NC * _NS


def _deg_call(dst_p, n):
    e_pad = dst_p.shape[0]
    _, n_pad = _npad(n)
    out_type = jax.ShapeDtypeStruct((_NC * _NS, n_pad), F32)
    scratch = [
        pltpu.VMEM((n_pad,), F32),
        pltpu.VMEM((e_pad // (_NC * _NS * 8),), I32),
    ]
    return pl.kernel(
        _deg_body, out_type=out_type, mesh=_mesh(), scratch_types=scratch,
        compiler_params=pltpu.CompilerParams(use_tc_tiling_on_sc=False, needs_layout_passes=False),
    )(dst_p)


def _degsum_body(dp_hbm, out_hbm, abuf, sbuf):
    c = lax.axis_index("c")
    t = lax.axis_index("s")
    w = c * _NS + t
    sl = abuf.shape[0]
    base = w * sl
    pltpu.sync_copy(dp_hbm.at[0, pl.ds(base, sl)], abuf)
    for k in range(1, _NC * _NS):
        pltpu.sync_copy(dp_hbm.at[k, pl.ds(base, sl)], sbuf)

        def rbody(i, carry):
            abuf[pl.ds(i * _L, _L)] = (abuf[pl.ds(i * _L, _L)]
                                       + sbuf[pl.ds(i * _L, _L)])
            return carry

        lax.fori_loop(0, sl // _L, rbody, 0, unroll=8)
    pltpu.sync_copy(abuf, out_hbm.at[pl.ds(base, sl)])


def _degsum_call(dp, n):
    sl, n_pad = _npad(n)
    out_type = jax.ShapeDtypeStruct((n_pad,), F32)
    scratch = [pltpu.VMEM((sl,), F32), pltpu.VMEM((sl,), F32)]
    return pl.kernel(
        _degsum_body, out_type=out_type, mesh=_mesh(), scratch_types=scratch,
        compiler_params=pltpu.CompilerParams(use_tc_tiling_on_sc=False, needs_layout_passes=False),
    )(dp)


def _prop_body(hs_hbm, edges_hbm, out_hbm,
               zbuf, ebuf_a, ebuf_b, rbuf_a, rbuf_b, isem, gsem, ssem, acc):
    c = lax.axis_index("c")
    t = lax.axis_index("s")
    n = out_hbm.shape[1]
    rows_tile = n // _NS
    zrows = zbuf.shape[0]
    z = jnp.zeros((_L,), F32)

    def zb(i, carry):
        zbuf[i, :] = z
        return carry

    lax.fori_loop(0, zrows, zb, 0, unroll=8)

    row0 = t * rows_tile
    for k in range(rows_tile // zrows):
        pltpu.sync_copy(zbuf, acc.at[pl.ds(row0 + k * zrows, zrows), :])
    plsc.subcore_barrier()

    tbl = hs_hbm.at[c]
    erows_tile = edges_hbm.shape[0] // _NS
    nbody = erows_tile // _PAIR
    rbase0 = t * erows_tile

    def body(i, carry):
        ra = rbase0 + i * _PAIR
        rb = ra + _CROWS

        @pl.when(i == 0)
        def _():
            pltpu.async_copy(edges_hbm.at[pl.ds(ra, _CROWS)], ebuf_a, isem)

        pltpu.make_async_copy(
            edges_hbm.at[pl.ds(ra, _CROWS)], ebuf_a, isem).wait()
        gda = [
            pltpu.async_copy(tbl.at[ebuf_a.at[j, 0]], rbuf_a.at[j], gsem)
            for j in range(_CROWS)
        ]
        db = pltpu.async_copy(edges_hbm.at[pl.ds(rb, _CROWS)], ebuf_b, isem)
        for d in gda:
            d.wait()
        sda = [
            pltpu.async_copy(rbuf_a.at[j], acc.at[ebuf_a.at[j, 1]], ssem,
                             add=True)
            for j in range(_CROWS)
        ]
        db.wait()
        gdb = [
            pltpu.async_copy(tbl.at[ebuf_b.at[j, 0]], rbuf_b.at[j], gsem)
            for j in range(_CROWS)
        ]
        for d in sda:
            d.wait()

        @pl.when(i < nbody - 1)
        def _():
            pltpu.async_copy(
                edges_hbm.at[pl.ds(ra + _PAIR, _CROWS)], ebuf_a, isem)

        for d in gdb:
            d.wait()
        sdb = [
            pltpu.async_copy(rbuf_b.at[j], acc.at[ebuf_b.at[j, 1]], ssem,
                             add=True)
            for j in range(_CROWS)
        ]
        for d in sdb:
            d.wait()
        return carry

    lax.fori_loop(0, nbody, body, 0)
    plsc.subcore_barrier()
    pltpu.sync_copy(acc.at[pl.ds(row0, rows_tile), :],
                    out_hbm.at[c, pl.ds(row0, rows_tile), :])


def _prop_call(hs, edges, n):
    out_type = jax.ShapeDtypeStruct((_NC, n, _L), F32)
    scratch = [
        pltpu.VMEM((n // _NS // 25, _L), F32),
        pltpu.VMEM((_CROWS, 2, _ROW), I32),
        pltpu.VMEM((_CROWS, 2, _ROW), I32),
        pltpu.VMEM((_CROWS, _ROW, _L), F32),
        pltpu.VMEM((_CROWS, _ROW, _L), F32),
        pltpu.SemaphoreType.DMA,
        pltpu.SemaphoreType.DMA,
        pltpu.SemaphoreType.DMA,
        pltpu.VMEM_SHARED((n + _L, _L), F32),
    ]
    return pl.kernel(
        _prop_body, out_type=out_type, mesh=_mesh(), scratch_types=scratch,
        compiler_params=pltpu.CompilerParams(
            use_tc_tiling_on_sc=False, needs_layout_passes=False,
            internal_scratch_in_bytes=131072),
    )(hs, edges)


def _tc_first(x, dp, w1, n):
    def body(x_ref, dp_ref, w_ref, hs_ref, dinv_ref):
        deg = dp_ref[...]
        dinv = jnp.where(deg > 0, lax.rsqrt(jnp.maximum(deg, 1.0)), 0.0)
        h = jnp.dot(x_ref[...], w_ref[...], preferred_element_type=F32)
        hs = h * dinv
        hs_ref[...] = jnp.stack([hs[:, :_L], hs[:, _L:]], axis=0)
        dinv_ref[...] = dinv

    return pl.pallas_call(
        body,
        grid=(n // _BN,),
        in_specs=[
            pl.BlockSpec((_BN, 5), lambda i: (i, 0)),
            pl.BlockSpec((_BN, 1), lambda i: (i, 0)),
            pl.BlockSpec((5, 32), lambda i: (0, 0)),
        ],
        out_specs=[
            pl.BlockSpec((_NC, _BN, _L), lambda i: (0, i, 0)),
            pl.BlockSpec((_BN, 1), lambda i: (i, 0)),
        ],
        out_shape=[
            jax.ShapeDtypeStruct((_NC, n, _L), F32),
            jax.ShapeDtypeStruct((n, 1), F32),
        ],
    )(x, dp, w1)


def _tc_mid(prop, dinv, b, w, n):
    def body(p_ref, dinv_ref, b_ref, w_ref, hs_ref):
        pm = jnp.concatenate([p_ref[0], p_ref[1]], axis=1)
        di = dinv_ref[...]
        h = jnp.maximum(pm * di + b_ref[...], 0.0)
        hs = jnp.dot(h, w_ref[...], preferred_element_type=F32) * di
        hs_ref[...] = jnp.stack([hs[:, :_L], hs[:, _L:]], axis=0)

    return pl.pallas_call(
        body,
        grid=(n // _BN,),
        in_specs=[
            pl.BlockSpec((_NC, _BN, _L), lambda i: (0, i, 0)),
            pl.BlockSpec((_BN, 1), lambda i: (i, 0)),
            pl.BlockSpec((1, 32), lambda i: (0, 0)),
            pl.BlockSpec((32, 32), lambda i: (0, 0)),
        ],
        out_specs=pl.BlockSpec((_NC, _BN, _L), lambda i: (0, i, 0)),
        out_shape=jax.ShapeDtypeStruct((_NC, n, _L), F32),
    )(prop, dinv, b, w)


def _tc_final(prop, dinv, b3, wdc, bdc, wta, bta, n):
    def body(p_ref, dinv_ref, b_ref, wdc_ref, bdc_ref, wta_ref, bta_ref,
             dist_ref, chase_ref, tls_ref, al_ref, smax_ref, srow_ref):
        i = pl.program_id(0)
        pm = jnp.concatenate([p_ref[0], p_ref[1]], axis=1)
        di = dinv_ref[...]
        h = jnp.maximum(pm * di + b_ref[...], 0.0)
        dc = jnp.dot(h, wdc_ref[...], preferred_element_type=F32) + bdc_ref[...]
        dist_ref[...] = dc[:, 0:1]
        chase_ref[...] = dc[:, 1:2]

        @pl.when(i == 0)
        def _():
            smax_ref[0] = -jnp.inf

        chase = dc[:, 1]
        bm = jnp.max(chase)

        @pl.when(bm > smax_ref[0])
        def _():
            smax_ref[0] = bm
            am = jnp.argmax(chase)
            sel = (lax.broadcasted_iota(I32, (chase.shape[0], 1), 0)
                   == am).astype(F32)
            srow_ref[...] = jnp.sum(h * sel, axis=0, keepdims=True)

        @pl.when(i == pl.num_programs(0) - 1)
        def _():
            ht = srow_ref[...]
            ta = (jnp.dot(ht, wta_ref[...], preferred_element_type=F32)
                  + bta_ref[...])
            tls_ref[...] = ta[:, 0:2]
            al_ref[...] = ta[:, 2:11]

    return pl.pallas_call(
        body,
        grid=(n // _BN,),
        in_specs=[
            pl.BlockSpec((_NC, _BN, _L), lambda i: (0, i, 0)),
            pl.BlockSpec((_BN, 1), lambda i: (i, 0)),
            pl.BlockSpec((1, 32), lambda i: (0, 0)),
            pl.BlockSpec((32, 2), lambda i: (0, 0)),
            pl.BlockSpec((1, 2), lambda i: (0, 0)),
            pl.BlockSpec((32, 11), lambda i: (0, 0)),
            pl.BlockSpec((1, 11), lambda i: (0, 0)),
        ],
        out_specs=[
            pl.BlockSpec((_BN, 1), lambda i: (i, 0)),
            pl.BlockSpec((_BN, 1), lambda i: (i, 0)),
            pl.BlockSpec((1, 2), lambda i: (0, 0)),
            pl.BlockSpec((1, 9), lambda i: (0, 0)),
        ],
        out_shape=[
            jax.ShapeDtypeStruct((n, 1), F32),
            jax.ShapeDtypeStruct((n, 1), F32),
            jax.ShapeDtypeStruct((1, 2), F32),
            jax.ShapeDtypeStruct((1, 9), F32),
        ],
        scratch_shapes=[
            pltpu.SMEM((1,), F32),
            pltpu.VMEM((1, 32), F32),
        ],
    )(prop, dinv, b3, wdc, bdc, wta, bta)


def kernel(x, edge_index, W1, b1, W2, b2, W3, b3,
           Wd, bd, Wc, bc, Wt, bt, Wa, ba):
    n = x.shape[0]
    e = edge_index.shape[1]
    unit = _NS * _ROW * _PAIR
    e_pad = (e + unit - 1) // unit * unit
    pad = e_pad - e

    src = edge_index[0]
    dst = edge_index[1]
    src_p = jnp.concatenate([src, jnp.zeros((pad,), I32)])
    dst_p = jnp.concatenate([dst, jnp.full((pad,), n, I32)])
    edges = jnp.stack(
        [src_p.reshape(-1, _ROW), dst_p.reshape(-1, _ROW)], axis=1)

    dpart = _deg_call(dst_p, n)
    deg = _degsum_call(dpart, n)
    hs1, dinv = _tc_first(x, deg[:n].reshape(n, 1), W1, n)
    p1 = _prop_call(hs1, edges, n)
    hs2 = _tc_mid(p1, dinv, b1.reshape(1, 32), W2, n)
    p2 = _prop_call(hs2, edges, n)
    hs3 = _tc_mid(p2, dinv, b2.reshape(1, 32), W3, n)
    p3 = _prop_call(hs3, edges, n)

    wdc = jnp.concatenate([Wd, Wc], axis=1)
    bdc = jnp.concatenate([bd, bc]).reshape(1, 2)
    wta = jnp.concatenate([Wt, Wa], axis=1)
    bta = jnp.concatenate([bt, ba]).reshape(1, 11)
    dist, chase, tls, al = _tc_final(
        p3, dinv, b3.reshape(1, 32), wdc, bdc, wta, bta, n)
    return (dist.reshape(n), chase.reshape(n),
            tls.reshape(2), al.reshape(9))
